# baseline (speedup 1.0000x reference)
import functools

import numpy as np
import jax
import jax.numpy as jnp
from jax import lax
from jax.experimental import pallas as pl
from jax.experimental.pallas import tpu as pltpu
from jax.experimental.pallas import tpu_sc as plsc

L_SPHER = 7
N_SPHER = 6
CUTOFF = 5.0
ENVELOPE_P = 6
N_COLS = L_SPHER * N_SPHER


def _jn_np(r, n):
    r = np.asarray(r, dtype=np.float64)
    j0 = np.sin(r) / r
    if n == 0:
        return j0
    j1 = np.sin(r) / r**2 - np.cos(r) / r
    jm, jc = j0, j1
    for l in range(1, n):
        jm, jc = jc, (2 * l + 1) / r * jc - jm
    return jc


def _bisect(f, a, b, iters=100):
    fa = f(a)
    for _ in range(iters):
        m = 0.5 * (a + b)
        fm = f(m)
        if fa * fm <= 0:
            b = m
        else:
            a, fa = m, fm
    return 0.5 * (a + b)


def _jn_zeros(n, k):
    zerosj = np.zeros((n, k))
    zerosj[0] = np.arange(1, k + 1) * np.pi
    points = np.arange(1, k + n) * np.pi
    for i in range(1, n):
        racines = np.zeros(len(points) - 1)
        for j in range(len(points) - 1):
            racines[j] = _bisect(lambda r: _jn_np(r, i), points[j], points[j + 1])
        points = racines
        zerosj[i, :k] = racines[:k]
    return zerosj


_ZEROS = _jn_zeros(L_SPHER, N_SPHER)
_NORM = np.zeros((L_SPHER, N_SPHER))
for _l in range(L_SPHER):
    for _n in range(N_SPHER):
        _NORM[_l, _n] = 1.0 / np.sqrt(0.5 * _jn_np(_ZEROS[_l, _n], _l + 1) ** 2)

_ZARG_ROW = np.float32(
    np.asarray(_ZEROS, np.float32) * np.float32(0.2)
).reshape(1, N_COLS)
_NORM_ROW = np.asarray(_NORM, np.float32).reshape(1, N_COLS)
_CBF_ROW = np.repeat(
    np.asarray(
        [float(np.sqrt((2 * l + 1) / (4.0 * np.pi))) for l in range(L_SPHER)],
        np.float32,
    ),
    N_SPHER,
).reshape(1, N_COLS)


def _sc_gather(d, idx):
    info = plsc.get_sparse_core_info()
    nw = info.num_cores * info.num_subcores
    b = idx.shape[0]
    bpw = b // nw
    mesh = plsc.VectorSubcoreMesh(core_axis_name="c", subcore_axis_name="s")

    @functools.partial(
        pl.kernel,
        out_type=jax.ShapeDtypeStruct((b,), jnp.float32),
        mesh=mesh,
        scratch_types=[
            pltpu.VMEM((bpw,), jnp.int32),
            pltpu.VMEM((bpw,), jnp.float32),
            pltpu.SemaphoreType.DMA,
        ],
    )
    def gat(d_hbm, idx_hbm, out_hbm, idx_v, rows_v, sem):
        wid = lax.axis_index("s") * info.num_cores + lax.axis_index("c")
        base = wid * bpw
        pltpu.sync_copy(idx_hbm.at[pl.ds(base, bpw)], idx_v)
        pltpu.async_copy(d_hbm.at[idx_v], rows_v, sem).wait()
        pltpu.sync_copy(rows_v, out_hbm.at[pl.ds(base, bpw)])

    return gat(d, idx)


_Z56 = np.ones((8 * L_SPHER, 1), np.float32)
_N56 = np.ones((8 * L_SPHER, 1), np.float32)
for _l in range(L_SPHER):
    _Z56[8 * _l : 8 * _l + N_SPHER, 0] = _ZARG_ROW[0, 6 * _l : 6 * _l + N_SPHER]
    _N56[8 * _l : 8 * _l + N_SPHER, 0] = _NORM_ROW[0, 6 * _l : 6 * _l + N_SPHER]
_CBF_L = [float(0.5 / np.sqrt(np.pi))] + [
    float(np.sqrt((2 * l + 1) / (4.0 * np.pi))) for l in range(1, L_SPHER)
]


def _tc_body(dg_ref, ang_ref, z56_ref, n56_ref, out_ref):
    d = dg_ref[...]
    ang = ang_ref[...]

    ds = d * np.float32(0.2)
    ds2 = ds * ds
    ds4 = ds2 * ds2
    ds6 = ds2 * ds4
    ds7 = (ds * ds2) * ds4
    ds8 = ds4 * ds4
    u = (1 - 28.0 * ds6 + 48.0 * ds7) - 21.0 * ds8

    ca = jnp.cos(ang)
    g = [None] * L_SPHER
    g[0] = np.float32(_CBF_L[0]) * u
    p0 = jnp.ones_like(ca)
    p1 = ca
    g[1] = (np.float32(_CBF_L[1]) * p1) * u
    for ll in range(1, L_SPHER - 1):
        p0, p1 = p1, ((2 * ll + 1) * ca * p1 - ll * p0) / (ll + 1)
        g[ll + 1] = (np.float32(_CBF_L[ll + 1]) * p1) * u

    z56 = z56_ref[...]
    n56 = n56_ref[...]
    for l in range(L_SPHER):
        z = z56[8 * l : 8 * l + N_SPHER]
        x = z * d
        s = jnp.sin(x)
        j0 = s / x
        if l == 0:
            jc = j0
        else:
            c = jnp.cos(x)
            j1 = s / (x * x) - c / x
            jm, jc = j0, j1
            for ll in range(1, l):
                jm, jc = jc, (2 * ll + 1) / x * jc - jm
        piece = (n56[8 * l : 8 * l + N_SPHER] * jc) * g[l]
        out_ref[pl.ds(6 * l, N_SPHER), :] = piece


def _tc_compute(dg, angles):
    n = dg.shape[0]
    eb = 6400
    grid = n // eb
    col_spec = pl.BlockSpec((8 * L_SPHER, 1), lambda i: (0, 0))
    out_t = pl.pallas_call(
        _tc_body,
        grid=(grid,),
        in_specs=[
            pl.BlockSpec((1, eb), lambda i: (0, i)),
            pl.BlockSpec((1, eb), lambda i: (0, i)),
            col_spec,
            col_spec,
        ],
        out_specs=pl.BlockSpec((N_COLS, eb), lambda i: (0, i)),
        out_shape=jax.ShapeDtypeStruct((N_COLS, n), jnp.float32),
    )(
        dg.reshape(1, n),
        angles.reshape(1, n),
        jnp.asarray(_Z56),
        jnp.asarray(_N56),
    )
    return out_t.T


def kernel(d, angles, kj_idx):
    dg = _sc_gather(d, kj_idx)
    return _tc_compute(dg, angles)


# speedup vs baseline: 1.0266x; 1.0266x over previous
import functools

import numpy as np
import jax
import jax.numpy as jnp
from jax import lax
from jax.experimental import pallas as pl
from jax.experimental.pallas import tpu as pltpu
from jax.experimental.pallas import tpu_sc as plsc

L_SPHER = 7
N_SPHER = 6
CUTOFF = 5.0
ENVELOPE_P = 6
N_COLS = L_SPHER * N_SPHER


def _jn_np(r, n):
    r = np.asarray(r, dtype=np.float64)
    j0 = np.sin(r) / r
    if n == 0:
        return j0
    j1 = np.sin(r) / r**2 - np.cos(r) / r
    jm, jc = j0, j1
    for l in range(1, n):
        jm, jc = jc, (2 * l + 1) / r * jc - jm
    return jc


def _bisect(f, a, b, iters=100):
    fa = f(a)
    for _ in range(iters):
        m = 0.5 * (a + b)
        fm = f(m)
        if fa * fm <= 0:
            b = m
        else:
            a, fa = m, fm
    return 0.5 * (a + b)


def _jn_zeros(n, k):
    zerosj = np.zeros((n, k))
    zerosj[0] = np.arange(1, k + 1) * np.pi
    points = np.arange(1, k + n) * np.pi
    for i in range(1, n):
        racines = np.zeros(len(points) - 1)
        for j in range(len(points) - 1):
            racines[j] = _bisect(lambda r: _jn_np(r, i), points[j], points[j + 1])
        points = racines
        zerosj[i, :k] = racines[:k]
    return zerosj


_ZEROS = _jn_zeros(L_SPHER, N_SPHER)
_NORM = np.zeros((L_SPHER, N_SPHER))
for _l in range(L_SPHER):
    for _n in range(N_SPHER):
        _NORM[_l, _n] = 1.0 / np.sqrt(0.5 * _jn_np(_ZEROS[_l, _n], _l + 1) ** 2)

_ZARG_ROW = np.float32(
    np.asarray(_ZEROS, np.float32) * np.float32(0.2)
).reshape(1, N_COLS)
_NORM_ROW = np.asarray(_NORM, np.float32).reshape(1, N_COLS)
_CBF_ROW = np.repeat(
    np.asarray(
        [float(np.sqrt((2 * l + 1) / (4.0 * np.pi))) for l in range(L_SPHER)],
        np.float32,
    ),
    N_SPHER,
).reshape(1, N_COLS)


def _sc_gather(d, idx):
    info = plsc.get_sparse_core_info()
    nw = info.num_cores * info.num_subcores
    b = idx.shape[0]
    bpw = b // nw
    mesh = plsc.VectorSubcoreMesh(core_axis_name="c", subcore_axis_name="s")

    @functools.partial(
        pl.kernel,
        out_type=jax.ShapeDtypeStruct((b,), jnp.float32),
        mesh=mesh,
        scratch_types=[
            pltpu.VMEM((bpw,), jnp.int32),
            pltpu.VMEM((bpw,), jnp.float32),
            pltpu.SemaphoreType.DMA,
        ],
    )
    def gat(d_hbm, idx_hbm, out_hbm, idx_v, rows_v, sem):
        wid = lax.axis_index("s") * info.num_cores + lax.axis_index("c")
        base = wid * bpw
        pltpu.sync_copy(idx_hbm.at[pl.ds(base, bpw)], idx_v)
        pltpu.async_copy(d_hbm.at[idx_v], rows_v, sem).wait()
        pltpu.sync_copy(rows_v, out_hbm.at[pl.ds(base, bpw)])

    return gat(d, idx)


_Z56 = np.ones((8 * L_SPHER, 1), np.float32)
_N56 = np.ones((8 * L_SPHER, 1), np.float32)
for _l in range(L_SPHER):
    _Z56[8 * _l : 8 * _l + N_SPHER, 0] = _ZARG_ROW[0, 6 * _l : 6 * _l + N_SPHER]
    _N56[8 * _l : 8 * _l + N_SPHER, 0] = _NORM_ROW[0, 6 * _l : 6 * _l + N_SPHER]
_CBF_L = [float(0.5 / np.sqrt(np.pi))] + [
    float(np.sqrt((2 * l + 1) / (4.0 * np.pi))) for l in range(1, L_SPHER)
]


def _tc_body(dg_ref, ang_ref, z56_ref, n56_ref, out_ref):
    d = dg_ref[...]
    ang = ang_ref[...]

    ds = d * np.float32(0.2)
    ds2 = ds * ds
    ds4 = ds2 * ds2
    ds6 = ds2 * ds4
    ds7 = (ds * ds2) * ds4
    ds8 = ds4 * ds4
    u = (1 - 28.0 * ds6 + 48.0 * ds7) - 21.0 * ds8

    ca = jnp.cos(ang)
    g = [None] * L_SPHER
    g[0] = np.float32(_CBF_L[0]) * u
    p0 = jnp.ones_like(ca)
    p1 = ca
    g[1] = (np.float32(_CBF_L[1]) * p1) * u
    for ll in range(1, L_SPHER - 1):
        p0, p1 = p1, ((2 * ll + 1) * ca * p1 - ll * p0) / (ll + 1)
        g[ll + 1] = (np.float32(_CBF_L[ll + 1]) * p1) * u

    z56 = z56_ref[...]
    n56 = n56_ref[...]
    for l in range(L_SPHER):
        z = z56[8 * l : 8 * l + N_SPHER]
        x = z * d
        s = jnp.sin(x)
        j0 = s / x
        if l == 0:
            jc = j0
        else:
            c = jnp.cos(x)
            j1 = s / (x * x) - c / x
            jm, jc = j0, j1
            for ll in range(1, l):
                jm, jc = jc, (2 * ll + 1) / x * jc - jm
        piece = (n56[8 * l : 8 * l + N_SPHER] * jc) * g[l]
        out_ref[pl.ds(6 * l, N_SPHER), :] = piece


def _tc_compute(dg, angles):
    n = dg.shape[0]
    eb = 2560
    grid = n // eb
    col_spec = pl.BlockSpec((8 * L_SPHER, 1), lambda i: (0, 0))
    out_t = pl.pallas_call(
        _tc_body,
        grid=(grid,),
        in_specs=[
            pl.BlockSpec((1, eb), lambda i: (0, i)),
            pl.BlockSpec((1, eb), lambda i: (0, i)),
            col_spec,
            col_spec,
        ],
        out_specs=pl.BlockSpec((N_COLS, eb), lambda i: (0, i)),
        out_shape=jax.ShapeDtypeStruct((N_COLS, n), jnp.float32),
    )(
        dg.reshape(1, n),
        angles.reshape(1, n),
        jnp.asarray(_Z56),
        jnp.asarray(_N56),
    )
    return out_t.T


def kernel(d, angles, kj_idx):
    dg = _sc_gather(d, kj_idx)
    return _tc_compute(dg, angles)


# speedup vs baseline: 1.0293x; 1.0026x over previous
import functools

import numpy as np
import jax
import jax.numpy as jnp
from jax import lax
from jax.experimental import pallas as pl
from jax.experimental.pallas import tpu as pltpu
from jax.experimental.pallas import tpu_sc as plsc

L_SPHER = 7
N_SPHER = 6
CUTOFF = 5.0
ENVELOPE_P = 6
N_COLS = L_SPHER * N_SPHER


def _jn_np(r, n):
    r = np.asarray(r, dtype=np.float64)
    j0 = np.sin(r) / r
    if n == 0:
        return j0
    j1 = np.sin(r) / r**2 - np.cos(r) / r
    jm, jc = j0, j1
    for l in range(1, n):
        jm, jc = jc, (2 * l + 1) / r * jc - jm
    return jc


def _bisect(f, a, b, iters=100):
    fa = f(a)
    for _ in range(iters):
        m = 0.5 * (a + b)
        fm = f(m)
        if fa * fm <= 0:
            b = m
        else:
            a, fa = m, fm
    return 0.5 * (a + b)


def _jn_zeros(n, k):
    zerosj = np.zeros((n, k))
    zerosj[0] = np.arange(1, k + 1) * np.pi
    points = np.arange(1, k + n) * np.pi
    for i in range(1, n):
        racines = np.zeros(len(points) - 1)
        for j in range(len(points) - 1):
            racines[j] = _bisect(lambda r: _jn_np(r, i), points[j], points[j + 1])
        points = racines
        zerosj[i, :k] = racines[:k]
    return zerosj


_ZEROS = _jn_zeros(L_SPHER, N_SPHER)
_NORM = np.zeros((L_SPHER, N_SPHER))
for _l in range(L_SPHER):
    for _n in range(N_SPHER):
        _NORM[_l, _n] = 1.0 / np.sqrt(0.5 * _jn_np(_ZEROS[_l, _n], _l + 1) ** 2)

_ZARG_ROW = np.float32(
    np.asarray(_ZEROS, np.float32) * np.float32(0.2)
).reshape(1, N_COLS)
_NORM_ROW = np.asarray(_NORM, np.float32).reshape(1, N_COLS)
_CBF_ROW = np.repeat(
    np.asarray(
        [float(np.sqrt((2 * l + 1) / (4.0 * np.pi))) for l in range(L_SPHER)],
        np.float32,
    ),
    N_SPHER,
).reshape(1, N_COLS)


def _sc_gather(d, idx):
    info = plsc.get_sparse_core_info()
    nw = info.num_cores * info.num_subcores
    b = idx.shape[0]
    bpw = b // nw
    mesh = plsc.VectorSubcoreMesh(core_axis_name="c", subcore_axis_name="s")

    @functools.partial(
        pl.kernel,
        out_type=jax.ShapeDtypeStruct((b,), jnp.float32),
        mesh=mesh,
        scratch_types=[
            pltpu.VMEM((bpw,), jnp.int32),
            pltpu.VMEM((bpw,), jnp.float32),
            pltpu.SemaphoreType.DMA,
        ],
    )
    def gat(d_hbm, idx_hbm, out_hbm, idx_v, rows_v, sem):
        wid = lax.axis_index("s") * info.num_cores + lax.axis_index("c")
        base = wid * bpw
        pltpu.sync_copy(idx_hbm.at[pl.ds(base, bpw)], idx_v)
        pltpu.async_copy(d_hbm.at[idx_v], rows_v, sem).wait()
        pltpu.sync_copy(rows_v, out_hbm.at[pl.ds(base, bpw)])

    return gat(d, idx)


_Z56 = np.ones((8 * L_SPHER, 1), np.float32)
_N56 = np.ones((8 * L_SPHER, 1), np.float32)
for _l in range(L_SPHER):
    _Z56[8 * _l : 8 * _l + N_SPHER, 0] = _ZARG_ROW[0, 6 * _l : 6 * _l + N_SPHER]
    _N56[8 * _l : 8 * _l + N_SPHER, 0] = _NORM_ROW[0, 6 * _l : 6 * _l + N_SPHER]
_CBF_L = [float(0.5 / np.sqrt(np.pi))] + [
    float(np.sqrt((2 * l + 1) / (4.0 * np.pi))) for l in range(1, L_SPHER)
]


def _tc_body(dg_ref, ang_ref, z56_ref, n56_ref, out_ref):
    d = dg_ref[...]
    ang = ang_ref[...]

    ds = d * np.float32(0.2)
    ds2 = ds * ds
    ds4 = ds2 * ds2
    ds6 = ds2 * ds4
    ds7 = (ds * ds2) * ds4
    ds8 = ds4 * ds4
    u = (1 - 28.0 * ds6 + 48.0 * ds7) - 21.0 * ds8

    ca = jnp.cos(ang)
    g = [None] * L_SPHER
    g[0] = np.float32(_CBF_L[0]) * u
    p0 = jnp.ones_like(ca)
    p1 = ca
    g[1] = (np.float32(_CBF_L[1]) * p1) * u
    for ll in range(1, L_SPHER - 1):
        p0, p1 = p1, ((2 * ll + 1) * ca * p1 - ll * p0) / (ll + 1)
        g[ll + 1] = (np.float32(_CBF_L[ll + 1]) * p1) * u

    z56 = z56_ref[...]
    n56 = n56_ref[...]
    for l in range(L_SPHER):
        z = z56[8 * l : 8 * l + N_SPHER]
        x = z * d
        s = jnp.sin(x)
        j0 = s / x
        if l == 0:
            jc = j0
        else:
            c = jnp.cos(x)
            j1 = s / (x * x) - c / x
            jm, jc = j0, j1
            for ll in range(1, l):
                jm, jc = jc, (2 * ll + 1) / x * jc - jm
        piece = (n56[8 * l : 8 * l + N_SPHER] * jc) * g[l]
        out_ref[pl.ds(6 * l, N_SPHER), :] = piece


def _tc_compute(dg, angles):
    n = dg.shape[0]
    eb = 3200
    grid = n // eb
    col_spec = pl.BlockSpec((8 * L_SPHER, 1), lambda i: (0, 0))
    out_t = pl.pallas_call(
        _tc_body,
        grid=(grid,),
        in_specs=[
            pl.BlockSpec((1, eb), lambda i: (0, i)),
            pl.BlockSpec((1, eb), lambda i: (0, i)),
            col_spec,
            col_spec,
        ],
        out_specs=pl.BlockSpec((N_COLS, eb), lambda i: (0, i)),
        out_shape=jax.ShapeDtypeStruct((N_COLS, n), jnp.float32),
    )(
        dg.reshape(1, n),
        angles.reshape(1, n),
        jnp.asarray(_Z56),
        jnp.asarray(_N56),
    )
    return out_t.T


def kernel(d, angles, kj_idx):
    dg = _sc_gather(d, kj_idx)
    return _tc_compute(dg, angles)


# speedup vs baseline: 1.0433x; 1.0136x over previous
import functools

import numpy as np
import jax
import jax.numpy as jnp
from jax import lax
from jax.experimental import pallas as pl
from jax.experimental.pallas import tpu as pltpu
from jax.experimental.pallas import tpu_sc as plsc

L_SPHER = 7
N_SPHER = 6
CUTOFF = 5.0
ENVELOPE_P = 6
N_COLS = L_SPHER * N_SPHER


def _jn_np(r, n):
    r = np.asarray(r, dtype=np.float64)
    j0 = np.sin(r) / r
    if n == 0:
        return j0
    j1 = np.sin(r) / r**2 - np.cos(r) / r
    jm, jc = j0, j1
    for l in range(1, n):
        jm, jc = jc, (2 * l + 1) / r * jc - jm
    return jc


def _bisect(f, a, b, iters=100):
    fa = f(a)
    for _ in range(iters):
        m = 0.5 * (a + b)
        fm = f(m)
        if fa * fm <= 0:
            b = m
        else:
            a, fa = m, fm
    return 0.5 * (a + b)


def _jn_zeros(n, k):
    zerosj = np.zeros((n, k))
    zerosj[0] = np.arange(1, k + 1) * np.pi
    points = np.arange(1, k + n) * np.pi
    for i in range(1, n):
        racines = np.zeros(len(points) - 1)
        for j in range(len(points) - 1):
            racines[j] = _bisect(lambda r: _jn_np(r, i), points[j], points[j + 1])
        points = racines
        zerosj[i, :k] = racines[:k]
    return zerosj


_ZEROS = _jn_zeros(L_SPHER, N_SPHER)
_NORM = np.zeros((L_SPHER, N_SPHER))
for _l in range(L_SPHER):
    for _n in range(N_SPHER):
        _NORM[_l, _n] = 1.0 / np.sqrt(0.5 * _jn_np(_ZEROS[_l, _n], _l + 1) ** 2)

_ZARG_ROW = np.float32(
    np.asarray(_ZEROS, np.float32) * np.float32(0.2)
).reshape(1, N_COLS)
_NORM_ROW = np.asarray(_NORM, np.float32).reshape(1, N_COLS)


def _sc_gather(d, idx):
    info = plsc.get_sparse_core_info()
    nw = info.num_cores * info.num_subcores
    b = idx.shape[0]
    bpw = b // nw
    mesh = plsc.VectorSubcoreMesh(core_axis_name="c", subcore_axis_name="s")

    @functools.partial(
        pl.kernel,
        out_type=jax.ShapeDtypeStruct((b,), jnp.float32),
        mesh=mesh,
        scratch_types=[
            pltpu.VMEM((bpw,), jnp.int32),
            pltpu.VMEM((bpw,), jnp.float32),
            pltpu.SemaphoreType.DMA,
        ],
    )
    def gat(d_hbm, idx_hbm, out_hbm, idx_v, rows_v, sem):
        wid = lax.axis_index("s") * info.num_cores + lax.axis_index("c")
        base = wid * bpw
        pltpu.sync_copy(idx_hbm.at[pl.ds(base, bpw)], idx_v)
        pltpu.async_copy(d_hbm.at[idx_v], rows_v, sem).wait()
        pltpu.sync_copy(rows_v, out_hbm.at[pl.ds(base, bpw)])

    return gat(d, idx)


_Z56 = np.ones((8 * L_SPHER, 1), np.float32)
_N56 = np.ones((8 * L_SPHER, 1), np.float32)
for _l in range(L_SPHER):
    _Z56[8 * _l : 8 * _l + N_SPHER, 0] = _ZARG_ROW[0, 6 * _l : 6 * _l + N_SPHER]
    _N56[8 * _l : 8 * _l + N_SPHER, 0] = _NORM_ROW[0, 6 * _l : 6 * _l + N_SPHER]
_CBF_L = [float(0.5 / np.sqrt(np.pi))] + [
    float(np.sqrt((2 * l + 1) / (4.0 * np.pi))) for l in range(1, L_SPHER)
]


def _tc_body(dg_ref, ang_ref, z56_ref, n56_ref, out_ref):
    d = dg_ref[...]
    ang = ang_ref[...]

    ds = d * np.float32(0.2)
    ds2 = ds * ds
    ds4 = ds2 * ds2
    ds6 = ds2 * ds4
    ds7 = (ds * ds2) * ds4
    ds8 = ds4 * ds4
    u = (1 - 28.0 * ds6 + 48.0 * ds7) - 21.0 * ds8

    ca = jnp.cos(ang)
    g = [None] * L_SPHER
    g[0] = np.float32(_CBF_L[0]) * u
    p0 = jnp.ones_like(ca)
    p1 = ca
    g[1] = (np.float32(_CBF_L[1]) * p1) * u
    for ll in range(1, L_SPHER - 1):
        p0, p1 = p1, ((2 * ll + 1) * ca * p1 - ll * p0) / (ll + 1)
        g[ll + 1] = (np.float32(_CBF_L[ll + 1]) * p1) * u

    z56 = z56_ref[...]
    n56 = n56_ref[...]
    for l in range(L_SPHER):
        z = z56[8 * l : 8 * l + N_SPHER]
        x = z * d
        s = jnp.sin(x)
        j0 = s / x
        if l == 0:
            jc = j0
        else:
            c = jnp.cos(x)
            j1 = s / (x * x) - c / x
            jm, jc = j0, j1
            for ll in range(1, l):
                jm, jc = jc, (2 * ll + 1) / x * jc - jm
        piece = (n56[8 * l : 8 * l + N_SPHER] * jc) * g[l]
        out_ref[pl.ds(6 * l, N_SPHER), :] = piece


_EB = 3200


def _tc_body2(dg_ref, ang_ref, z56_ref, n56_ref, prev_ref, out_ref):
    del prev_ref
    _tc_body(dg_ref, ang_ref, z56_ref, n56_ref, out_ref)


def _tc_piece(dg_half, angles_half, n_total, block_off, prev=None):
    h = dg_half.shape[0]
    grid = h // _EB
    col_spec = pl.BlockSpec((8 * L_SPHER, 1), lambda i: (0, 0))
    in_specs = [
        pl.BlockSpec((1, _EB), lambda i: (0, i)),
        pl.BlockSpec((1, _EB), lambda i: (0, i)),
        col_spec,
        col_spec,
    ]
    args = [
        dg_half.reshape(1, h),
        angles_half.reshape(1, h),
        jnp.asarray(_Z56),
        jnp.asarray(_N56),
    ]
    kwargs = {}
    if prev is None:
        body = _tc_body
    else:
        body = _tc_body2
        in_specs.append(pl.BlockSpec(memory_space=pl.ANY))
        args.append(prev)
        kwargs["input_output_aliases"] = {4: 0}
    return pl.pallas_call(
        body,
        grid=(grid,),
        in_specs=in_specs,
        out_specs=pl.BlockSpec(
            (N_COLS, _EB), lambda i, _o=block_off: (0, i + _o)
        ),
        out_shape=jax.ShapeDtypeStruct((N_COLS, n_total), jnp.float32),
        **kwargs,
    )(*args)


def kernel(d, angles, kj_idx):
    n = kj_idx.shape[0]
    h = n // 2
    dg_a = _sc_gather(d, kj_idx[:h])
    dg_b = _sc_gather(d, kj_idx[h:])
    out = _tc_piece(dg_a, angles[:h], n, 0)
    out = _tc_piece(dg_b, angles[h:], n, h // _EB, prev=out)
    return out.T
